# Initial kernel scaffold; baseline (speedup 1.0000x reference)
import jax
import jax.numpy as jnp
from jax.experimental import pallas as pl


def kernel(x, n_ids, ei0, ei1, RL_thresholds, W1, att_s1, att_d1, b1, bn_g, bn_b, W2, att_s2, att_d2, b2, Wm1, bm1, gm, betam, Wm2, bm2):
    raise NotImplementedError("write your pallas kernel here")


# speedup vs baseline: 21.5367x; 21.5367x over previous
import functools

import jax
import jax.numpy as jnp
from jax import lax
from jax.experimental import pallas as pl
from jax.experimental.pallas import tpu as pltpu
from jax.experimental.pallas import tpu_sc as plsc

N_GLOBAL = 100000
D_IN = 128
HID = 64
OUT = 64
HEADS = 4
R = 3
N_T0 = 10000
N_T1 = 2048
E0 = 320000
E1 = 32768

F32 = jnp.float32
I32 = jnp.int32

_info = plsc.get_sparse_core_info()
NC, NS, L = _info.num_cores, _info.num_subcores, _info.num_lanes
NW = NC * NS

_MESH = plsc.VectorSubcoreMesh(core_axis_name="c", subcore_axis_name="s")


GROWS = 960
GHALF = 480
GCHUNK = 120
NGPAD = NW * GROWS


@functools.partial(
    pl.kernel,
    out_type=jax.ShapeDtypeStruct((NGPAD, D_IN), F32),
    mesh=_MESH,
    compiler_params=pltpu.CompilerParams(needs_layout_passes=False),
    scratch_types=[
        pltpu.VMEM((GROWS,), I32),
        pltpu.VMEM((GHALF, D_IN), F32),
        pltpu.SemaphoreType.DMA,
    ],
)
def _gather_rows(x_hbm, idx_hbm, out_hbm, idx_v, rows_v, sem):
    wid = lax.axis_index("s") * NC + lax.axis_index("c")
    base = wid * GROWS
    pltpu.sync_copy(idx_hbm.at[pl.ds(base, GROWS)], idx_v)
    for g in range(GROWS // GHALF):
        for j in range(GHALF // GCHUNK):
            pltpu.async_copy(
                x_hbm.at[idx_v.at[pl.ds(g * GHALF + j * GCHUNK, GCHUNK)]],
                rows_v.at[pl.ds(j * GCHUNK, GCHUNK)],
                sem,
            )
        for j in range(GHALF // GCHUNK):
            pltpu.make_async_copy(
                x_hbm.at[idx_v.at[pl.ds(g * GHALF + j * GCHUNK, GCHUNK)]],
                rows_v.at[pl.ds(j * GCHUNK, GCHUNK)],
                sem,
            ).wait()
        pltpu.sync_copy(rows_v, out_hbm.at[pl.ds(base + g * GHALF, GHALF)])


BROWS = 1000
NB_B = N_T0 // BROWS


def _mm1_body(xs_ref, w1_ref, as_ref, ad_ref, h_ref, als_ref, ald_ref,
              ms_ref, md_ref):
    b = pl.program_id(1)
    x = xs_ref[0]
    w = w1_ref[0]
    h = jnp.dot(x, w, preferred_element_type=F32)
    h_ref[0, 0] = h[:, :128]
    h_ref[1, 0] = h[:, 128:]
    hh = h.reshape(BROWS, HEADS, HID)
    als = (hh * as_ref[0][None]).sum(-1)
    ald = (hh * ad_ref[0][None]).sum(-1)
    als_ref[0] = als
    ald_ref[0] = ald
    pad = jnp.full((12,), -1e30, F32)
    cs = jnp.concatenate([als.max(0), pad])
    cd = jnp.concatenate([ald.max(0), pad])

    @pl.when(b == 0)
    def _():
        ms_ref[0, 0] = cs
        md_ref[0, 0] = cd

    @pl.when(b > 0)
    def _():
        ms_ref[0, 0] = jnp.maximum(ms_ref[0, 0], cs)
        md_ref[0, 0] = jnp.maximum(md_ref[0, 0], cd)


def _mm1(xs3, W1, att_s1, att_d1):
    return pl.pallas_call(
        _mm1_body,
        grid=(R, NB_B),
        in_specs=[
            pl.BlockSpec((1, BROWS, D_IN), lambda r, b: (r, b, 0)),
            pl.BlockSpec((1, D_IN, HEADS * HID), lambda r, b: (r, 0, 0)),
            pl.BlockSpec((1, HEADS, HID), lambda r, b: (r, 0, 0)),
            pl.BlockSpec((1, HEADS, HID), lambda r, b: (r, 0, 0)),
        ],
        out_specs=[
            pl.BlockSpec((2, 1, BROWS, 128), lambda r, b: (0, r, b, 0)),
            pl.BlockSpec((1, BROWS, HEADS), lambda r, b: (r, b, 0)),
            pl.BlockSpec((1, BROWS, HEADS), lambda r, b: (r, b, 0)),
            pl.BlockSpec((1, 1, 16), lambda r, b: (r, 0, 0)),
            pl.BlockSpec((1, 1, 16), lambda r, b: (r, 0, 0)),
        ],
        out_shape=[
            jax.ShapeDtypeStruct((2, R, N_T0, 128), F32),
            jax.ShapeDtypeStruct((R, N_T0, HEADS), F32),
            jax.ShapeDtypeStruct((R, N_T0, HEADS), F32),
            jax.ShapeDtypeStruct((R, 1, 16), F32),
            jax.ShapeDtypeStruct((R, 1, 16), F32),
        ],
    )(xs3, W1, att_s1, att_d1)


def _make_edge_kernel(n_tgt, n_edges, hl, cg, chunk, head_split, row_cols=None):
    cols = row_cols if row_cols is not None else hl * cg
    th = hl * 2 if head_split else hl
    ept = n_edges // NS
    nchunk = ept // chunk
    assert nchunk * chunk == ept
    stripe = (n_tgt // NS) & ~7
    tail = n_tgt - stripe * NS
    assert tail % 8 == 0 and tail <= chunk
    nsub = chunk // L

    def _stripe_chunks():
        off = 0
        rem = stripe
        while rem > 0:
            n = min(rem, chunk)
            yield off, n
            off += n
            rem -= n

    def body(h_hbm, alsf_hbm, aldf_hbm, ms_hbm, md_hbm, src_hbm, dst_hbm,
             out_hbm, exw_hbm, als_v, ald_v, m_v, src_v, dst_v, stage_v,
             denr_v, w_v, rows_v, out_sp, *den_sps):
        c = lax.axis_index("c")
        s = lax.axis_index("s")
        ebase = s * ept
        zv = jnp.zeros((L,), F32)

        for r in range(R):
            pltpu.sync_copy(ms_hbm.at[r], m_v.at[pl.ds(0, 1)])
            pltpu.sync_copy(md_hbm.at[r], m_v.at[pl.ds(1, 1)])
            m_v[0] = m_v[0] + m_v[1]

            @pl.loop(0, chunk)
            def _(k):
                for v in range(cols // L):
                    rows_v[k, pl.ds(v * L, L)] = zv

            @pl.loop(0, chunk // L)
            def _(k16):
                stage_v[pl.ds(k16 * L, L)] = zv

            for off, n in _stripe_chunks():
                pltpu.sync_copy(rows_v.at[pl.ds(0, n)],
                                out_sp.at[pl.ds(s * stripe + off, n)])
                for dsp in den_sps:
                    pltpu.sync_copy(stage_v.at[pl.ds(0, n)],
                                    dsp.at[pl.ds(s * stripe + off, n)])
            if tail:
                @pl.when(s == NS - 1)
                def _():
                    pltpu.sync_copy(rows_v.at[pl.ds(0, tail)],
                                    out_sp.at[pl.ds(NS * stripe, tail)])
                    for dsp in den_sps:
                        pltpu.sync_copy(stage_v.at[pl.ds(0, tail)],
                                        dsp.at[pl.ds(NS * stripe, tail)])
            plsc.subcore_barrier()

            for hli in range(hl):
                hc = c * hl + hli if head_split else hli
                al_base = (hc * R + r) * n_tgt
                pltpu.sync_copy(alsf_hbm.at[pl.ds(al_base, n_tgt)], als_v)
                pltpu.sync_copy(aldf_hbm.at[pl.ds(al_base, n_tgt)], ald_v)
                msp = plsc.load_gather(
                    m_v, [jnp.zeros((L,), I32), jnp.full((L,), hc, I32)])

                @pl.loop(0, nchunk)
                def _(ci):
                    e0 = ebase + ci * chunk
                    pltpu.sync_copy(
                        src_hbm.at[pl.ds(r * n_edges + e0, chunk)], src_v)
                    pltpu.sync_copy(
                        dst_hbm.at[pl.ds(r * n_edges + e0, chunk)], dst_v)
                    for j in range(nsub):
                        src16 = src_v[pl.ds(j * L, L)]
                        dst16 = dst_v[pl.ds(j * L, L)]
                        a = plsc.load_gather(als_v, [src16])
                        d = plsc.load_gather(ald_v, [dst16])
                        al = a + d
                        e = jnp.where(al > 0, al, al * F32(0.2))
                        stage_v[pl.ds(j * L, L)] = jnp.exp(e - msp)
                    pltpu.sync_copy(
                        stage_v, exw_hbm.at[pl.ds(hc * n_edges + e0, chunk)])
                    pltpu.sync_copy(stage_v, den_sps[hli].at[dst_v], add=True)

            plsc.subcore_barrier()

            @pl.loop(0, nchunk)
            def _(ci):
                e0 = ebase + ci * chunk
                pltpu.sync_copy(
                    src_hbm.at[pl.ds(r * n_edges + e0, chunk)], src_v)
                pltpu.sync_copy(
                    dst_hbm.at[pl.ds(r * n_edges + e0, chunk)], dst_v)
                pltpu.sync_copy(h_hbm.at[c, r].at[src_v], rows_v)
                for hli in range(hl):
                    hc = c * hl + hli if head_split else hli
                    pltpu.sync_copy(
                        exw_hbm.at[pl.ds(hc * n_edges + e0, chunk)], stage_v)
                    pltpu.sync_copy(den_sps[hli].at[dst_v], denr_v)
                    for j in range(nsub):
                        ex = stage_v[pl.ds(j * L, L)]
                        den = denr_v[pl.ds(j * L, L)]
                        w_v[pl.ds(j * L, L)] = ex / (den + F32(1e-16))

                    @pl.loop(0, chunk)
                    def _(k):
                        wv = plsc.load_gather(w_v, [jnp.full((L,), k, I32)])
                        for v in range(cg // L):
                            sl = pl.ds(hli * cg + v * L, L)
                            rows_v[k, sl] = rows_v[k, sl] * wv

                pltpu.sync_copy(rows_v, out_sp.at[dst_v], add=True)

            plsc.subcore_barrier()

            for off, n in _stripe_chunks():
                pltpu.sync_copy(out_sp.at[pl.ds(s * stripe + off, n)],
                                rows_v.at[pl.ds(0, n)])
                pltpu.sync_copy(rows_v.at[pl.ds(0, n)],
                                out_hbm.at[c, r].at[pl.ds(s * stripe + off, n)])
            if tail:
                @pl.when(s == NS - 1)
                def _():
                    pltpu.sync_copy(out_sp.at[pl.ds(NS * stripe, tail)],
                                    rows_v.at[pl.ds(0, tail)])
                    pltpu.sync_copy(
                        rows_v.at[pl.ds(0, tail)],
                        out_hbm.at[c, r].at[pl.ds(NS * stripe, tail)])
            plsc.subcore_barrier()

    kern = pl.kernel(
        body,
        out_type=[
            jax.ShapeDtypeStruct((2, R, n_tgt, cols), F32),
            jax.ShapeDtypeStruct((th * n_edges,), F32),
        ],
        mesh=_MESH,
        compiler_params=pltpu.CompilerParams(needs_layout_passes=False),
        scratch_types=[
            pltpu.VMEM((n_tgt,), F32),
            pltpu.VMEM((n_tgt,), F32),
            pltpu.VMEM((2, 16), F32),
            pltpu.VMEM((chunk,), I32),
            pltpu.VMEM((chunk,), I32),
            pltpu.VMEM((chunk,), F32),
            pltpu.VMEM((chunk,), F32),
            pltpu.VMEM((chunk,), F32),
            pltpu.VMEM((chunk, cols), F32),
            pltpu.VMEM_SHARED((n_tgt, cols), F32),
        ] + [pltpu.VMEM_SHARED((n_tgt,), F32) for _ in range(hl)],
    )
    return kern


_edge_l1 = _make_edge_kernel(N_T0, E0, 2, 64, 80, True)
_edge_l2 = _make_edge_kernel(N_T1, E1, 1, 32, 128, False, row_cols=128)


def _d1_body(o_ref, stats_ref):
    b = pl.program_id(1)
    a = o_ref[0, 0]
    bb = o_ref[1, 0]
    sa = a.sum(0)
    sb = bb.sum(0)
    qa = (a * a).sum(0)
    qb = (bb * bb).sum(0)
    st = jnp.concatenate(
        [sa[None], sb[None], qa[None], qb[None],
         jnp.zeros((4, 128), F32)], axis=0)

    @pl.when(b == 0)
    def _():
        stats_ref[0] = st

    @pl.when(b > 0)
    def _():
        stats_ref[0] = stats_ref[0] + st


def _d1(out1):
    return pl.pallas_call(
        _d1_body,
        grid=(R, NB_B),
        in_specs=[pl.BlockSpec((2, 1, BROWS, 128), lambda r, b: (0, r, b, 0))],
        out_specs=pl.BlockSpec((1, 8, 128), lambda r, b: (r, 0, 0)),
        out_shape=jax.ShapeDtypeStruct((R, 8, 128), F32),
    )(out1)


DROWS = 256
NB_D = N_T1 // DROWS


def _d2_body(o_ref, stats_ref, g_ref, bta_ref, w2_ref, as2_ref, ad2_ref,
             h2_ref, als2_ref, ald2_ref, ms2_ref, md2_ref):
    b = pl.program_id(1)
    x = jnp.concatenate([o_ref[0, 0], o_ref[1, 0]], axis=1)
    st = stats_ref[0]
    mu = jnp.concatenate([st[0], st[1]]) * F32(1.0 / N_T0)
    sq = jnp.concatenate([st[2], st[3]]) * F32(1.0 / N_T0)
    var = sq - mu * mu
    inv = lax.rsqrt(var + F32(1e-5))
    xn = g_ref[0, 0][None] * (x - mu[None]) * inv[None] + bta_ref[0, 0][None]
    h = jnp.where(xn > 0, xn, jnp.exp(xn) - F32(1.0))
    h2 = jnp.dot(h, w2_ref[0], preferred_element_type=F32)
    zpad = jnp.zeros((DROWS, 96), F32)
    h2_ref[0, 0] = jnp.concatenate([h2[:, :32], zpad], axis=1)
    h2_ref[1, 0] = jnp.concatenate([h2[:, 32:], zpad], axis=1)
    als = (h2 * as2_ref[0, 0][None]).sum(-1)
    ald = (h2 * ad2_ref[0, 0][None]).sum(-1)
    z7 = jnp.zeros((DROWS, 7), F32)
    als2_ref[0] = jnp.concatenate([als[:, None], z7], axis=1)
    ald2_ref[0] = jnp.concatenate([ald[:, None], z7], axis=1)
    pad = jnp.full((15,), -1e30, F32)
    cs = jnp.concatenate([als.max()[None], pad])
    cd = jnp.concatenate([ald.max()[None], pad])

    @pl.when(b == 0)
    def _():
        ms2_ref[0, 0] = cs
        md2_ref[0, 0] = cd

    @pl.when(b > 0)
    def _():
        ms2_ref[0, 0] = jnp.maximum(ms2_ref[0, 0], cs)
        md2_ref[0, 0] = jnp.maximum(md2_ref[0, 0], cd)


def _d2(out1, stats, bn_g, bn_b, W2, att_s2, att_d2):
    return pl.pallas_call(
        _d2_body,
        grid=(R, NB_D),
        in_specs=[
            pl.BlockSpec((2, 1, DROWS, 128), lambda r, b: (0, r, b, 0)),
            pl.BlockSpec((1, 8, 128), lambda r, b: (r, 0, 0)),
            pl.BlockSpec((1, 1, HEADS * HID), lambda r, b: (r, 0, 0)),
            pl.BlockSpec((1, 1, HEADS * HID), lambda r, b: (r, 0, 0)),
            pl.BlockSpec((1, HEADS * HID, OUT), lambda r, b: (r, 0, 0)),
            pl.BlockSpec((1, 1, OUT), lambda r, b: (r, 0, 0)),
            pl.BlockSpec((1, 1, OUT), lambda r, b: (r, 0, 0)),
        ],
        out_specs=[
            pl.BlockSpec((2, 1, DROWS, 128), lambda r, b: (0, r, b, 0)),
            pl.BlockSpec((1, DROWS, 8), lambda r, b: (r, b, 0)),
            pl.BlockSpec((1, DROWS, 8), lambda r, b: (r, b, 0)),
            pl.BlockSpec((1, 1, 16), lambda r, b: (r, 0, 0)),
            pl.BlockSpec((1, 1, 16), lambda r, b: (r, 0, 0)),
        ],
        out_shape=[
            jax.ShapeDtypeStruct((2, R, N_T1, 128), F32),
            jax.ShapeDtypeStruct((R, N_T1, 8), F32),
            jax.ShapeDtypeStruct((R, N_T1, 8), F32),
            jax.ShapeDtypeStruct((R, 1, 16), F32),
            jax.ShapeDtypeStruct((R, 1, 16), F32),
        ],
    )(out1, stats, bn_g, bn_b, W2, att_s2, att_d2)


FROWS = 256
NB_F = N_T1 // FROWS


def _f_body(o2_ref, b2f_ref, rlf_ref, wm1_ref, bm1_ref, gm_ref, btm_ref,
            wm2_ref, bm2_ref, out_ref, g_scr, st_scr):
    p = pl.program_id(0)
    b = pl.program_id(1)

    @pl.when(p == 0)
    def _():
        f = jnp.concatenate(
            [o2_ref[0, 0][:, :32], o2_ref[1, 0][:, :32],
             o2_ref[0, 1][:, :32], o2_ref[1, 1][:, :32],
             o2_ref[0, 2][:, :32], o2_ref[1, 2][:, :32]], axis=1)
        f = (f + b2f_ref[0][None]) * rlf_ref[0][None]
        g = jnp.dot(f, wm1_ref[...], preferred_element_type=F32) \
            + bm1_ref[0][None]
        g_scr[pl.ds(b * FROWS, FROWS)] = g
        st = jnp.concatenate(
            [g.sum(0)[None], (g * g).sum(0)[None],
             jnp.zeros((6, 192), F32)], axis=0)

        @pl.when(b == 0)
        def _():
            st_scr[...] = st

        @pl.when(b > 0)
        def _():
            st_scr[...] = st_scr[...] + st

    @pl.when(p == 1)
    def _():
        g = g_scr[pl.ds(b * FROWS, FROWS)]
        mu = st_scr[0] * F32(1.0 / N_T1)
        var = st_scr[1] * F32(1.0 / N_T1) - mu * mu
        inv = lax.rsqrt(var + F32(1e-5))
        gn = gm_ref[0][None] * (g - mu[None]) * inv[None] + btm_ref[0][None]
        gn = jnp.maximum(gn, F32(0.0))
        out_ref[...] = jnp.dot(gn, wm2_ref[...],
                               preferred_element_type=F32) + bm2_ref[0][None]


def _f_stage(out2, b2f, rlf, Wm1, bm1, gm, betam, Wm2, bm2):
    return pl.pallas_call(
        _f_body,
        grid=(2, NB_F),
        in_specs=[
            pl.BlockSpec((2, R, FROWS, 128), lambda p, b: (0, 0, b, 0)),
            pl.BlockSpec((1, R * OUT), lambda p, b: (0, 0)),
            pl.BlockSpec((1, R * OUT), lambda p, b: (0, 0)),
            pl.BlockSpec((R * OUT, R * OUT), lambda p, b: (0, 0)),
            pl.BlockSpec((1, R * OUT), lambda p, b: (0, 0)),
            pl.BlockSpec((1, R * OUT), lambda p, b: (0, 0)),
            pl.BlockSpec((1, R * OUT), lambda p, b: (0, 0)),
            pl.BlockSpec((R * OUT, OUT), lambda p, b: (0, 0)),
            pl.BlockSpec((1, OUT), lambda p, b: (0, 0)),
        ],
        out_specs=pl.BlockSpec((FROWS, OUT), lambda p, b: (b, 0)),
        out_shape=jax.ShapeDtypeStruct((N_T1, OUT), F32),
        scratch_shapes=[
            pltpu.VMEM((N_T1, R * OUT), F32),
            pltpu.VMEM((8, R * OUT), F32),
        ],
    )(out2, b2f, rlf, Wm1, bm1, gm, betam, Wm2, bm2)


def kernel(x, n_ids, ei0, ei1, RL_thresholds, W1, att_s1, att_d1, b1, bn_g,
           bn_b, W2, att_s2, att_d2, b2, Wm1, bm1, gm, betam, Wm2, bm2):
    idx = n_ids[:, :N_T0].reshape(-1).astype(I32)
    idx = jnp.concatenate([idx, jnp.zeros((NGPAD - R * N_T0,), I32)])
    src0 = ei0[:, 0, :].reshape(-1).astype(I32)
    dst0 = ei0[:, 1, :].reshape(-1).astype(I32)
    src1 = ei1[:, 0, :].reshape(-1).astype(I32)
    dst1 = ei1[:, 1, :].reshape(-1).astype(I32)

    xs = _gather_rows(x, idx)
    xs3 = xs[: R * N_T0].reshape(R, N_T0, D_IN)

    H1, als1, ald1, ms1, md1 = _mm1(xs3, W1, att_s1, att_d1)

    alsf1 = als1.transpose(2, 0, 1).reshape(-1)
    aldf1 = ald1.transpose(2, 0, 1).reshape(-1)
    out1, _ = _edge_l1(H1, alsf1, aldf1, ms1, md1, src0, dst0)

    stats = _d1(out1)
    H2, als2, ald2, ms2, md2 = _d2(out1, stats, bn_g[:, None], bn_b[:, None],
                                   W2, att_s2, att_d2)

    alsf2 = als2[:, :, 0].reshape(-1)
    aldf2 = ald2[:, :, 0].reshape(-1)
    out2, _ = _edge_l2(H2, alsf2, aldf2, ms2, md2, src1, dst1)

    b2f = b2.reshape(1, R * OUT)
    rlf = jnp.repeat(RL_thresholds[:, 0], OUT).reshape(1, R * OUT)
    return _f_stage(out2, b2f, rlf, Wm1, bm1.reshape(1, -1), gm.reshape(1, -1),
                    betam.reshape(1, -1), Wm2, bm2.reshape(1, -1))


# speedup vs baseline: 35.8694x; 1.6655x over previous
import functools

import jax
import jax.numpy as jnp
from jax import lax
from jax.experimental import pallas as pl
from jax.experimental.pallas import tpu as pltpu
from jax.experimental.pallas import tpu_sc as plsc

N_GLOBAL = 100000
D_IN = 128
HID = 64
OUT = 64
HEADS = 4
R = 3
N_T0 = 10000
N_T1 = 2048
E0 = 320000
E1 = 32768

F32 = jnp.float32
I32 = jnp.int32

_info = plsc.get_sparse_core_info()
NC, NS, L = _info.num_cores, _info.num_subcores, _info.num_lanes
NW = NC * NS

_MESH = plsc.VectorSubcoreMesh(core_axis_name="c", subcore_axis_name="s")


GROWS = 960
GHALF = 480
GCHUNK = 120
NGPAD = NW * GROWS


@functools.partial(
    pl.kernel,
    out_type=jax.ShapeDtypeStruct((NGPAD, D_IN), F32),
    mesh=_MESH,
    compiler_params=pltpu.CompilerParams(needs_layout_passes=False),
    scratch_types=[
        pltpu.VMEM((GROWS,), I32),
        pltpu.VMEM((GHALF, D_IN), F32),
        pltpu.SemaphoreType.DMA,
    ],
)
def _gather_rows(x_hbm, idx_hbm, out_hbm, idx_v, rows_v, sem):
    wid = lax.axis_index("s") * NC + lax.axis_index("c")
    base = wid * GROWS
    pltpu.sync_copy(idx_hbm.at[pl.ds(base, GROWS)], idx_v)
    for g in range(GROWS // GHALF):
        for j in range(GHALF // GCHUNK):
            pltpu.async_copy(
                x_hbm.at[idx_v.at[pl.ds(g * GHALF + j * GCHUNK, GCHUNK)]],
                rows_v.at[pl.ds(j * GCHUNK, GCHUNK)],
                sem,
            )
        for j in range(GHALF // GCHUNK):
            pltpu.make_async_copy(
                x_hbm.at[idx_v.at[pl.ds(g * GHALF + j * GCHUNK, GCHUNK)]],
                rows_v.at[pl.ds(j * GCHUNK, GCHUNK)],
                sem,
            ).wait()
        pltpu.sync_copy(rows_v, out_hbm.at[pl.ds(base + g * GHALF, GHALF)])


BROWS = 1000
NB_B = N_T0 // BROWS


def _mm1_body(xs_ref, w1_ref, as_ref, ad_ref, h_ref, als_ref, ald_ref,
              ms_ref, md_ref):
    b = pl.program_id(1)
    x = xs_ref[0]
    w = w1_ref[0]
    h = jnp.dot(x, w, preferred_element_type=F32)
    h_ref[0, 0] = h[:, :128]
    h_ref[1, 0] = h[:, 128:]
    hh = h.reshape(BROWS, HEADS, HID)
    als = (hh * as_ref[0][None]).sum(-1)
    ald = (hh * ad_ref[0][None]).sum(-1)
    als_ref[0] = als
    ald_ref[0] = ald
    pad = jnp.full((12,), -1e30, F32)
    cs = jnp.concatenate([als.max(0), pad])
    cd = jnp.concatenate([ald.max(0), pad])

    @pl.when(b == 0)
    def _():
        ms_ref[0, 0] = cs
        md_ref[0, 0] = cd

    @pl.when(b > 0)
    def _():
        ms_ref[0, 0] = jnp.maximum(ms_ref[0, 0], cs)
        md_ref[0, 0] = jnp.maximum(md_ref[0, 0], cd)


def _mm1(xs3, W1, att_s1, att_d1):
    return pl.pallas_call(
        _mm1_body,
        grid=(R, NB_B),
        in_specs=[
            pl.BlockSpec((1, BROWS, D_IN), lambda r, b: (r, b, 0)),
            pl.BlockSpec((1, D_IN, HEADS * HID), lambda r, b: (r, 0, 0)),
            pl.BlockSpec((1, HEADS, HID), lambda r, b: (r, 0, 0)),
            pl.BlockSpec((1, HEADS, HID), lambda r, b: (r, 0, 0)),
        ],
        out_specs=[
            pl.BlockSpec((2, 1, BROWS, 128), lambda r, b: (0, r, b, 0)),
            pl.BlockSpec((1, BROWS, HEADS), lambda r, b: (r, b, 0)),
            pl.BlockSpec((1, BROWS, HEADS), lambda r, b: (r, b, 0)),
            pl.BlockSpec((1, 1, 16), lambda r, b: (r, 0, 0)),
            pl.BlockSpec((1, 1, 16), lambda r, b: (r, 0, 0)),
        ],
        out_shape=[
            jax.ShapeDtypeStruct((2, R, N_T0, 128), F32),
            jax.ShapeDtypeStruct((R, N_T0, HEADS), F32),
            jax.ShapeDtypeStruct((R, N_T0, HEADS), F32),
            jax.ShapeDtypeStruct((R, 1, 16), F32),
            jax.ShapeDtypeStruct((R, 1, 16), F32),
        ],
    )(xs3, W1, att_s1, att_d1)


def _make_edge_kernel(n_tgt, n_edges, hl, cg, head_split, row_cols=None):
    cols = row_cols if row_cols is not None else hl * cg
    th = hl * 2 if head_split else hl
    ept = n_edges // NS
    chunk = 128
    nchunk = ept // chunk
    etail = ept - nchunk * chunk
    assert etail % 16 == 0
    stripe = (n_tgt // NS) & ~7
    tail = n_tgt - stripe * NS
    assert tail % 8 == 0 and tail <= chunk

    def _stripe_chunks():
        off = 0
        rem = stripe
        while rem > 0:
            n = min(rem, chunk)
            yield off, n
            off += n
            rem -= n

    def body(h_hbm, alsf_hbm, aldf_hbm, ms_hbm, md_hbm, src_hbm, dst_hbm,
             out_hbm, exw_hbm, als_v, ald_v, m_v, src_v, dst_v, dst_t,
             stage_v, stage2_v, denr_v, denr2_v, w_v, rows_v, sem0, sem1,
             out_sp, *den_sps):
        c = lax.axis_index("c")
        s = lax.axis_index("s")
        ebase = s * ept
        zv = jnp.zeros((L,), F32)
        stages = (stage_v, stage2_v)
        denrs = (denr_v, denr2_v)

        def pa_chunk(r, hli, hc, msp, e0, csz, dref):
            d1 = pltpu.async_copy(src_hbm.at[pl.ds(r * n_edges + e0, csz)],
                                  src_v.at[pl.ds(0, csz)], sem0)
            d2 = pltpu.async_copy(dst_hbm.at[pl.ds(r * n_edges + e0, csz)],
                                  dref, sem0)
            d1.wait()
            d2.wait()
            for j in range(csz // L):
                src16 = src_v[pl.ds(j * L, L)]
                dst16 = dref[pl.ds(j * L, L)] if csz == chunk \
                    else dst_t[pl.ds(j * L, L)]
                a = plsc.load_gather(als_v, [src16])
                d = plsc.load_gather(ald_v, [dst16])
                al = a + d
                e = jnp.where(al > 0, al, al * F32(0.2))
                stage_v[pl.ds(j * L, L)] = jnp.exp(e - msp)
            d3 = pltpu.async_copy(
                stage_v.at[pl.ds(0, csz)],
                exw_hbm.at[pl.ds(hc * n_edges + e0, csz)], sem1)
            pltpu.sync_copy(stage_v.at[pl.ds(0, csz)],
                            den_sps[hli].at[dref], add=True)
            d3.wait()

        def pb_chunk(r, e0, csz, dref):
            d1 = pltpu.async_copy(src_hbm.at[pl.ds(r * n_edges + e0, csz)],
                                  src_v.at[pl.ds(0, csz)], sem0)
            d2 = pltpu.async_copy(dst_hbm.at[pl.ds(r * n_edges + e0, csz)],
                                  dref, sem0)
            exds = []
            for hli in range(hl):
                hc = c * hl + hli if head_split else hli
                exds.append(pltpu.async_copy(
                    exw_hbm.at[pl.ds(hc * n_edges + e0, csz)],
                    stages[hli].at[pl.ds(0, csz)], sem0))
            d1.wait()
            d2.wait()
            pltpu.sync_copy(h_hbm.at[c, r].at[src_v.at[pl.ds(0, csz)]],
                            rows_v.at[pl.ds(0, csz)])
            for hli in range(hl):
                pltpu.sync_copy(den_sps[hli].at[dref],
                                denrs[hli].at[pl.ds(0, csz)])
            for d in exds:
                d.wait()
            for hli in range(hl):
                for j in range(csz // L):
                    ex = stages[hli][pl.ds(j * L, L)]
                    den = denrs[hli][pl.ds(j * L, L)]
                    w_v[pl.ds(j * L, L)] = ex / (den + F32(1e-16))

                @pl.loop(0, csz)
                def _(k):
                    wv = plsc.load_gather(w_v, [jnp.full((L,), k, I32)])
                    for v in range(cg // L):
                        sl = pl.ds(hli * cg + v * L, L)
                        rows_v[k, sl] = rows_v[k, sl] * wv

            pltpu.sync_copy(rows_v.at[pl.ds(0, csz)], out_sp.at[dref],
                            add=True)

        for r in range(R):
            pltpu.sync_copy(ms_hbm.at[r], m_v.at[pl.ds(0, 1)])
            pltpu.sync_copy(md_hbm.at[r], m_v.at[pl.ds(1, 1)])
            m_v[0] = m_v[0] + m_v[1]

            @pl.loop(0, chunk)
            def _(k):
                for v in range(cols // L):
                    rows_v[k, pl.ds(v * L, L)] = zv

            @pl.loop(0, chunk // L)
            def _(k16):
                stage_v[pl.ds(k16 * L, L)] = zv

            for off, n in _stripe_chunks():
                pltpu.sync_copy(rows_v.at[pl.ds(0, n)],
                                out_sp.at[pl.ds(s * stripe + off, n)])
                for dsp in den_sps:
                    pltpu.sync_copy(stage_v.at[pl.ds(0, n)],
                                    dsp.at[pl.ds(s * stripe + off, n)])
            if tail:
                @pl.when(s == NS - 1)
                def _():
                    pltpu.sync_copy(rows_v.at[pl.ds(0, tail)],
                                    out_sp.at[pl.ds(NS * stripe, tail)])
                    for dsp in den_sps:
                        pltpu.sync_copy(stage_v.at[pl.ds(0, tail)],
                                        dsp.at[pl.ds(NS * stripe, tail)])
            plsc.subcore_barrier()

            for hli in range(hl):
                hc = c * hl + hli if head_split else hli
                al_base = (hc * R + r) * n_tgt
                pltpu.sync_copy(alsf_hbm.at[pl.ds(al_base, n_tgt)], als_v)
                pltpu.sync_copy(aldf_hbm.at[pl.ds(al_base, n_tgt)], ald_v)
                msp = plsc.load_gather(
                    m_v, [jnp.zeros((L,), I32), jnp.full((L,), hc, I32)])

                @pl.loop(0, nchunk)
                def _(ci):
                    pa_chunk(r, hli, hc, msp, ebase + ci * chunk, chunk,
                             dst_v)
                if etail:
                    pa_chunk(r, hli, hc, msp, ebase + nchunk * chunk,
                             etail, dst_t)

            plsc.subcore_barrier()

            @pl.loop(0, nchunk)
            def _(ci):
                pb_chunk(r, ebase + ci * chunk, chunk, dst_v)
            if etail:
                pb_chunk(r, ebase + nchunk * chunk, etail, dst_t)

            plsc.subcore_barrier()

            for off, n in _stripe_chunks():
                pltpu.sync_copy(out_sp.at[pl.ds(s * stripe + off, n)],
                                rows_v.at[pl.ds(0, n)])
                pltpu.sync_copy(rows_v.at[pl.ds(0, n)],
                                out_hbm.at[c, r].at[pl.ds(s * stripe + off, n)])
            if tail:
                @pl.when(s == NS - 1)
                def _():
                    pltpu.sync_copy(out_sp.at[pl.ds(NS * stripe, tail)],
                                    rows_v.at[pl.ds(0, tail)])
                    pltpu.sync_copy(
                        rows_v.at[pl.ds(0, tail)],
                        out_hbm.at[c, r].at[pl.ds(NS * stripe, tail)])
            plsc.subcore_barrier()

    kern = pl.kernel(
        body,
        out_type=[
            jax.ShapeDtypeStruct((2, R, n_tgt, cols), F32),
            jax.ShapeDtypeStruct((th * n_edges,), F32),
        ],
        mesh=_MESH,
        compiler_params=pltpu.CompilerParams(needs_layout_passes=False),
        scratch_types=[
            pltpu.VMEM((n_tgt,), F32),
            pltpu.VMEM((n_tgt,), F32),
            pltpu.VMEM((2, 16), F32),
            pltpu.VMEM((128,), I32),
            pltpu.VMEM((128,), I32),
            pltpu.VMEM((32,), I32),
            pltpu.VMEM((128,), F32),
            pltpu.VMEM((128,), F32),
            pltpu.VMEM((128,), F32),
            pltpu.VMEM((128,), F32),
            pltpu.VMEM((128,), F32),
            pltpu.VMEM((128, cols), F32),
            pltpu.SemaphoreType.DMA,
            pltpu.SemaphoreType.DMA,
            pltpu.VMEM_SHARED((n_tgt, cols), F32),
        ] + [pltpu.VMEM_SHARED((n_tgt,), F32) for _ in range(hl)],
    )
    return kern


_edge_l1 = _make_edge_kernel(N_T0, E0, 2, 64, True)
_edge_l2 = _make_edge_kernel(N_T1, E1, 1, 32, False, row_cols=128)


def _d1_body(o_ref, stats_ref):
    b = pl.program_id(1)
    a = o_ref[0, 0]
    bb = o_ref[1, 0]
    sa = a.sum(0)
    sb = bb.sum(0)
    qa = (a * a).sum(0)
    qb = (bb * bb).sum(0)
    st = jnp.concatenate(
        [sa[None], sb[None], qa[None], qb[None],
         jnp.zeros((4, 128), F32)], axis=0)

    @pl.when(b == 0)
    def _():
        stats_ref[0] = st

    @pl.when(b > 0)
    def _():
        stats_ref[0] = stats_ref[0] + st


def _d1(out1):
    return pl.pallas_call(
        _d1_body,
        grid=(R, NB_B),
        in_specs=[pl.BlockSpec((2, 1, BROWS, 128), lambda r, b: (0, r, b, 0))],
        out_specs=pl.BlockSpec((1, 8, 128), lambda r, b: (r, 0, 0)),
        out_shape=jax.ShapeDtypeStruct((R, 8, 128), F32),
    )(out1)


DROWS = 256
NB_D = N_T1 // DROWS


def _d2_body(o_ref, stats_ref, g_ref, bta_ref, w2_ref, as2_ref, ad2_ref,
             h2_ref, als2_ref, ald2_ref, ms2_ref, md2_ref):
    b = pl.program_id(1)
    x = jnp.concatenate([o_ref[0, 0], o_ref[1, 0]], axis=1)
    st = stats_ref[0]
    mu = jnp.concatenate([st[0], st[1]]) * F32(1.0 / N_T0)
    sq = jnp.concatenate([st[2], st[3]]) * F32(1.0 / N_T0)
    var = sq - mu * mu
    inv = lax.rsqrt(var + F32(1e-5))
    xn = g_ref[0, 0][None] * (x - mu[None]) * inv[None] + bta_ref[0, 0][None]
    h = jnp.where(xn > 0, xn, jnp.exp(xn) - F32(1.0))
    h2 = jnp.dot(h, w2_ref[0], preferred_element_type=F32)
    zpad = jnp.zeros((DROWS, 96), F32)
    h2_ref[0, 0] = jnp.concatenate([h2[:, :32], zpad], axis=1)
    h2_ref[1, 0] = jnp.concatenate([h2[:, 32:], zpad], axis=1)
    als = (h2 * as2_ref[0, 0][None]).sum(-1)
    ald = (h2 * ad2_ref[0, 0][None]).sum(-1)
    z7 = jnp.zeros((DROWS, 7), F32)
    als2_ref[0] = jnp.concatenate([als[:, None], z7], axis=1)
    ald2_ref[0] = jnp.concatenate([ald[:, None], z7], axis=1)
    pad = jnp.full((15,), -1e30, F32)
    cs = jnp.concatenate([als.max()[None], pad])
    cd = jnp.concatenate([ald.max()[None], pad])

    @pl.when(b == 0)
    def _():
        ms2_ref[0, 0] = cs
        md2_ref[0, 0] = cd

    @pl.when(b > 0)
    def _():
        ms2_ref[0, 0] = jnp.maximum(ms2_ref[0, 0], cs)
        md2_ref[0, 0] = jnp.maximum(md2_ref[0, 0], cd)


def _d2(out1, stats, bn_g, bn_b, W2, att_s2, att_d2):
    return pl.pallas_call(
        _d2_body,
        grid=(R, NB_D),
        in_specs=[
            pl.BlockSpec((2, 1, DROWS, 128), lambda r, b: (0, r, b, 0)),
            pl.BlockSpec((1, 8, 128), lambda r, b: (r, 0, 0)),
            pl.BlockSpec((1, 1, HEADS * HID), lambda r, b: (r, 0, 0)),
            pl.BlockSpec((1, 1, HEADS * HID), lambda r, b: (r, 0, 0)),
            pl.BlockSpec((1, HEADS * HID, OUT), lambda r, b: (r, 0, 0)),
            pl.BlockSpec((1, 1, OUT), lambda r, b: (r, 0, 0)),
            pl.BlockSpec((1, 1, OUT), lambda r, b: (r, 0, 0)),
        ],
        out_specs=[
            pl.BlockSpec((2, 1, DROWS, 128), lambda r, b: (0, r, b, 0)),
            pl.BlockSpec((1, DROWS, 8), lambda r, b: (r, b, 0)),
            pl.BlockSpec((1, DROWS, 8), lambda r, b: (r, b, 0)),
            pl.BlockSpec((1, 1, 16), lambda r, b: (r, 0, 0)),
            pl.BlockSpec((1, 1, 16), lambda r, b: (r, 0, 0)),
        ],
        out_shape=[
            jax.ShapeDtypeStruct((2, R, N_T1, 128), F32),
            jax.ShapeDtypeStruct((R, N_T1, 8), F32),
            jax.ShapeDtypeStruct((R, N_T1, 8), F32),
            jax.ShapeDtypeStruct((R, 1, 16), F32),
            jax.ShapeDtypeStruct((R, 1, 16), F32),
        ],
    )(out1, stats, bn_g, bn_b, W2, att_s2, att_d2)


FROWS = 256
NB_F = N_T1 // FROWS


def _f_body(o2_ref, b2f_ref, rlf_ref, wm1_ref, bm1_ref, gm_ref, btm_ref,
            wm2_ref, bm2_ref, out_ref, g_scr, st_scr):
    p = pl.program_id(0)
    b = pl.program_id(1)

    @pl.when(p == 0)
    def _():
        f = jnp.concatenate(
            [o2_ref[0, 0][:, :32], o2_ref[1, 0][:, :32],
             o2_ref[0, 1][:, :32], o2_ref[1, 1][:, :32],
             o2_ref[0, 2][:, :32], o2_ref[1, 2][:, :32]], axis=1)
        f = (f + b2f_ref[0][None]) * rlf_ref[0][None]
        g = jnp.dot(f, wm1_ref[...], preferred_element_type=F32) \
            + bm1_ref[0][None]
        g_scr[pl.ds(b * FROWS, FROWS)] = g
        st = jnp.concatenate(
            [g.sum(0)[None], (g * g).sum(0)[None],
             jnp.zeros((6, 192), F32)], axis=0)

        @pl.when(b == 0)
        def _():
            st_scr[...] = st

        @pl.when(b > 0)
        def _():
            st_scr[...] = st_scr[...] + st

    @pl.when(p == 1)
    def _():
        g = g_scr[pl.ds(b * FROWS, FROWS)]
        mu = st_scr[0] * F32(1.0 / N_T1)
        var = st_scr[1] * F32(1.0 / N_T1) - mu * mu
        inv = lax.rsqrt(var + F32(1e-5))
        gn = gm_ref[0][None] * (g - mu[None]) * inv[None] + btm_ref[0][None]
        gn = jnp.maximum(gn, F32(0.0))
        out_ref[...] = jnp.dot(gn, wm2_ref[...],
                               preferred_element_type=F32) + bm2_ref[0][None]


def _f_stage(out2, b2f, rlf, Wm1, bm1, gm, betam, Wm2, bm2):
    return pl.pallas_call(
        _f_body,
        grid=(2, NB_F),
        in_specs=[
            pl.BlockSpec((2, R, FROWS, 128), lambda p, b: (0, 0, b, 0)),
            pl.BlockSpec((1, R * OUT), lambda p, b: (0, 0)),
            pl.BlockSpec((1, R * OUT), lambda p, b: (0, 0)),
            pl.BlockSpec((R * OUT, R * OUT), lambda p, b: (0, 0)),
            pl.BlockSpec((1, R * OUT), lambda p, b: (0, 0)),
            pl.BlockSpec((1, R * OUT), lambda p, b: (0, 0)),
            pl.BlockSpec((1, R * OUT), lambda p, b: (0, 0)),
            pl.BlockSpec((R * OUT, OUT), lambda p, b: (0, 0)),
            pl.BlockSpec((1, OUT), lambda p, b: (0, 0)),
        ],
        out_specs=pl.BlockSpec((FROWS, OUT), lambda p, b: (b, 0)),
        out_shape=jax.ShapeDtypeStruct((N_T1, OUT), F32),
        scratch_shapes=[
            pltpu.VMEM((N_T1, R * OUT), F32),
            pltpu.VMEM((8, R * OUT), F32),
        ],
    )(out2, b2f, rlf, Wm1, bm1, gm, betam, Wm2, bm2)


def kernel(x, n_ids, ei0, ei1, RL_thresholds, W1, att_s1, att_d1, b1, bn_g,
           bn_b, W2, att_s2, att_d2, b2, Wm1, bm1, gm, betam, Wm2, bm2):
    idx = n_ids[:, :N_T0].reshape(-1).astype(I32)
    idx = jnp.concatenate([idx, jnp.zeros((NGPAD - R * N_T0,), I32)])
    src0 = ei0[:, 0, :].reshape(-1).astype(I32)
    dst0 = ei0[:, 1, :].reshape(-1).astype(I32)
    src1 = ei1[:, 0, :].reshape(-1).astype(I32)
    dst1 = ei1[:, 1, :].reshape(-1).astype(I32)

    xs = _gather_rows(x, idx)
    xs3 = xs[: R * N_T0].reshape(R, N_T0, D_IN)

    H1, als1, ald1, ms1, md1 = _mm1(xs3, W1, att_s1, att_d1)

    alsf1 = als1.transpose(2, 0, 1).reshape(-1)
    aldf1 = ald1.transpose(2, 0, 1).reshape(-1)
    out1, _ = _edge_l1(H1, alsf1, aldf1, ms1, md1, src0, dst0)

    stats = _d1(out1)
    H2, als2, ald2, ms2, md2 = _d2(out1, stats, bn_g[:, None], bn_b[:, None],
                                   W2, att_s2, att_d2)

    alsf2 = als2[:, :, 0].reshape(-1)
    aldf2 = ald2[:, :, 0].reshape(-1)
    out2, _ = _edge_l2(H2, alsf2, aldf2, ms2, md2, src1, dst1)

    b2f = b2.reshape(1, R * OUT)
    rlf = jnp.repeat(RL_thresholds[:, 0], OUT).reshape(1, R * OUT)
    return _f_stage(out2, b2f, rlf, Wm1, bm1.reshape(1, -1), gm.reshape(1, -1),
                    betam.reshape(1, -1), Wm2, bm2.reshape(1, -1))


# speedup vs baseline: 37.2292x; 1.0379x over previous
import functools

import jax
import jax.numpy as jnp
from jax import lax
from jax.experimental import pallas as pl
from jax.experimental.pallas import tpu as pltpu
from jax.experimental.pallas import tpu_sc as plsc

N_GLOBAL = 100000
D_IN = 128
HID = 64
OUT = 64
HEADS = 4
R = 3
N_T0 = 10000
N_T1 = 2048
E0 = 320000
E1 = 32768

F32 = jnp.float32
I32 = jnp.int32

_info = plsc.get_sparse_core_info()
NC, NS, L = _info.num_cores, _info.num_subcores, _info.num_lanes
NW = NC * NS

_MESH = plsc.VectorSubcoreMesh(core_axis_name="c", subcore_axis_name="s")


GROWS = 960
GHALF = 480
GCHUNK = 120
NGPAD = NW * GROWS


@functools.partial(
    pl.kernel,
    out_type=jax.ShapeDtypeStruct((NGPAD, D_IN), F32),
    mesh=_MESH,
    compiler_params=pltpu.CompilerParams(needs_layout_passes=False),
    scratch_types=[
        pltpu.VMEM((GROWS,), I32),
        pltpu.VMEM((GHALF, D_IN), F32),
        pltpu.SemaphoreType.DMA,
    ],
)
def _gather_rows(x_hbm, idx_hbm, out_hbm, idx_v, rows_v, sem):
    wid = lax.axis_index("s") * NC + lax.axis_index("c")
    base = wid * GROWS
    pltpu.sync_copy(idx_hbm.at[pl.ds(base, GROWS)], idx_v)
    for g in range(GROWS // GHALF):
        for j in range(GHALF // GCHUNK):
            pltpu.async_copy(
                x_hbm.at[idx_v.at[pl.ds(g * GHALF + j * GCHUNK, GCHUNK)]],
                rows_v.at[pl.ds(j * GCHUNK, GCHUNK)],
                sem,
            )
        for j in range(GHALF // GCHUNK):
            pltpu.make_async_copy(
                x_hbm.at[idx_v.at[pl.ds(g * GHALF + j * GCHUNK, GCHUNK)]],
                rows_v.at[pl.ds(j * GCHUNK, GCHUNK)],
                sem,
            ).wait()
        pltpu.sync_copy(rows_v, out_hbm.at[pl.ds(base + g * GHALF, GHALF)])


BROWS = 1000
NB_B = N_T0 // BROWS


def _mm1_body(xs_ref, w1_ref, as_ref, ad_ref, h_ref, als_ref, ald_ref,
              ms_ref, md_ref):
    b = pl.program_id(1)
    x = xs_ref[0]
    w = w1_ref[0]
    h = jnp.dot(x, w, preferred_element_type=F32)
    h_ref[0, 0] = h[:, :128]
    h_ref[1, 0] = h[:, 128:]
    hh = h.reshape(BROWS, HEADS, HID)
    als = (hh * as_ref[0][None]).sum(-1)
    ald = (hh * ad_ref[0][None]).sum(-1)
    als_ref[0] = als
    ald_ref[0] = ald
    pad = jnp.full((12,), -1e30, F32)
    cs = jnp.concatenate([als.max(0), pad])
    cd = jnp.concatenate([ald.max(0), pad])

    @pl.when(b == 0)
    def _():
        ms_ref[0, 0] = cs
        md_ref[0, 0] = cd

    @pl.when(b > 0)
    def _():
        ms_ref[0, 0] = jnp.maximum(ms_ref[0, 0], cs)
        md_ref[0, 0] = jnp.maximum(md_ref[0, 0], cd)


def _mm1(xs3, W1, att_s1, att_d1):
    return pl.pallas_call(
        _mm1_body,
        grid=(R, NB_B),
        in_specs=[
            pl.BlockSpec((1, BROWS, D_IN), lambda r, b: (r, b, 0)),
            pl.BlockSpec((1, D_IN, HEADS * HID), lambda r, b: (r, 0, 0)),
            pl.BlockSpec((1, HEADS, HID), lambda r, b: (r, 0, 0)),
            pl.BlockSpec((1, HEADS, HID), lambda r, b: (r, 0, 0)),
        ],
        out_specs=[
            pl.BlockSpec((2, 1, BROWS, 128), lambda r, b: (0, r, b, 0)),
            pl.BlockSpec((1, BROWS, HEADS), lambda r, b: (r, b, 0)),
            pl.BlockSpec((1, BROWS, HEADS), lambda r, b: (r, b, 0)),
            pl.BlockSpec((1, 1, 16), lambda r, b: (r, 0, 0)),
            pl.BlockSpec((1, 1, 16), lambda r, b: (r, 0, 0)),
        ],
        out_shape=[
            jax.ShapeDtypeStruct((2, R, N_T0, 128), F32),
            jax.ShapeDtypeStruct((R, N_T0, HEADS), F32),
            jax.ShapeDtypeStruct((R, N_T0, HEADS), F32),
            jax.ShapeDtypeStruct((R, 1, 16), F32),
            jax.ShapeDtypeStruct((R, 1, 16), F32),
        ],
    )(xs3, W1, att_s1, att_d1)


def _make_edge_kernel(n_tgt, n_edges, hl, cg, head_split, row_cols=None):
    cols = row_cols if row_cols is not None else hl * cg
    th = hl * 2 if head_split else hl
    ept = n_edges // NS
    chunk = 128
    nchunk = ept // chunk
    etail = ept - nchunk * chunk
    assert etail % 16 == 0
    stripe = (n_tgt // NS) & ~7
    tail = n_tgt - stripe * NS
    assert tail % 8 == 0 and tail <= chunk

    def _stripe_chunks():
        off = 0
        rem = stripe
        while rem > 0:
            n = min(rem, chunk)
            yield off, n
            off += n
            rem -= n

    def body(h_hbm, alsf_hbm, aldf_hbm, ms_hbm, md_hbm, src_hbm, dst_hbm,
             out_hbm, exw_hbm, als_v, ald_v, m_v, src_v, dst_v, dst_t,
             stage_v, stage2_v, denr_v, denr2_v, w_v, rows_v, sem0, sem1,
             out_sp, *den_sps):
        c = lax.axis_index("c")
        s = lax.axis_index("s")
        ebase = s * ept
        zv = jnp.zeros((L,), F32)
        stages = (stage_v, stage2_v)
        denrs = (denr_v, denr2_v)

        def pa_chunk(r, hli, hc, msp, e0, csz, dref):
            d1 = pltpu.async_copy(src_hbm.at[pl.ds(r * n_edges + e0, csz)],
                                  src_v.at[pl.ds(0, csz)], sem0)
            d2 = pltpu.async_copy(dst_hbm.at[pl.ds(r * n_edges + e0, csz)],
                                  dref, sem0)
            d1.wait()
            d2.wait()
            for j in range(csz // L):
                src16 = src_v[pl.ds(j * L, L)]
                dst16 = dref[pl.ds(j * L, L)] if csz == chunk \
                    else dst_t[pl.ds(j * L, L)]
                a = plsc.load_gather(als_v, [src16])
                d = plsc.load_gather(ald_v, [dst16])
                al = a + d
                e = jnp.where(al > 0, al, al * F32(0.2))
                stage_v[pl.ds(j * L, L)] = jnp.exp(e - msp)
            d3 = pltpu.async_copy(
                stage_v.at[pl.ds(0, csz)],
                exw_hbm.at[pl.ds(hc * n_edges + e0, csz)], sem1)
            pltpu.sync_copy(stage_v.at[pl.ds(0, csz)],
                            den_sps[hli].at[dref], add=True)
            d3.wait()

        def pb_chunk(r, e0, csz, dref):
            d1 = pltpu.async_copy(src_hbm.at[pl.ds(r * n_edges + e0, csz)],
                                  src_v.at[pl.ds(0, csz)], sem0)
            d2 = pltpu.async_copy(dst_hbm.at[pl.ds(r * n_edges + e0, csz)],
                                  dref, sem0)
            exds = []
            for hli in range(hl):
                hc = c * hl + hli if head_split else hli
                exds.append(pltpu.async_copy(
                    exw_hbm.at[pl.ds(hc * n_edges + e0, csz)],
                    stages[hli].at[pl.ds(0, csz)], sem0))
            d1.wait()
            d2.wait()
            pltpu.sync_copy(h_hbm.at[c, r].at[src_v.at[pl.ds(0, csz)]],
                            rows_v.at[pl.ds(0, csz)])
            for d in exds:
                d.wait()
            for hli in range(hl):
                @pl.loop(0, csz)
                def _(k):
                    wv = plsc.load_gather(stages[hli],
                                          [jnp.full((L,), k, I32)])
                    for v in range(cg // L):
                        sl = pl.ds(hli * cg + v * L, L)
                        rows_v[k, sl] = rows_v[k, sl] * wv

            pltpu.sync_copy(rows_v.at[pl.ds(0, csz)], out_sp.at[dref],
                            add=True)

        def wb_chunk(r, row0, n):
            pltpu.sync_copy(out_sp.at[pl.ds(row0, n)], rows_v.at[pl.ds(0, n)])
            for hli in range(hl):
                pltpu.sync_copy(den_sps[hli].at[pl.ds(row0, n)],
                                denrs[hli].at[pl.ds(0, n)])
                for j in range(n // L):
                    dsl = pl.ds(j * L, L)
                    denrs[hli][dsl] = jnp.full((L,), 1.0, F32) / (
                        denrs[hli][dsl] + F32(1e-16))

            @pl.loop(0, n)
            def _(k):
                kv = jnp.full((L,), k, I32)
                for hli in range(hl):
                    iv = plsc.load_gather(denrs[hli], [kv])
                    for v in range(cg // L):
                        sl = pl.ds(hli * cg + v * L, L)
                        rows_v[k, sl] = rows_v[k, sl] * iv

            pltpu.sync_copy(rows_v.at[pl.ds(0, n)],
                            out_hbm.at[c, r].at[pl.ds(row0, n)])

        for r in range(R):
            pltpu.sync_copy(ms_hbm.at[r], m_v.at[pl.ds(0, 1)])
            pltpu.sync_copy(md_hbm.at[r], m_v.at[pl.ds(1, 1)])
            m_v[0] = m_v[0] + m_v[1]

            @pl.loop(0, chunk)
            def _(k):
                for v in range(cols // L):
                    rows_v[k, pl.ds(v * L, L)] = zv

            @pl.loop(0, chunk // L)
            def _(k16):
                stage_v[pl.ds(k16 * L, L)] = zv

            for off, n in _stripe_chunks():
                pltpu.sync_copy(rows_v.at[pl.ds(0, n)],
                                out_sp.at[pl.ds(s * stripe + off, n)])
                for dsp in den_sps:
                    pltpu.sync_copy(stage_v.at[pl.ds(0, n)],
                                    dsp.at[pl.ds(s * stripe + off, n)])
            if tail:
                @pl.when(s == NS - 1)
                def _():
                    pltpu.sync_copy(rows_v.at[pl.ds(0, tail)],
                                    out_sp.at[pl.ds(NS * stripe, tail)])
                    for dsp in den_sps:
                        pltpu.sync_copy(stage_v.at[pl.ds(0, tail)],
                                        dsp.at[pl.ds(NS * stripe, tail)])
            plsc.subcore_barrier()

            for hli in range(hl):
                hc = c * hl + hli if head_split else hli
                al_base = (hc * R + r) * n_tgt
                pltpu.sync_copy(alsf_hbm.at[pl.ds(al_base, n_tgt)], als_v)
                pltpu.sync_copy(aldf_hbm.at[pl.ds(al_base, n_tgt)], ald_v)
                msp = plsc.load_gather(
                    m_v, [jnp.zeros((L,), I32), jnp.full((L,), hc, I32)])

                @pl.loop(0, nchunk)
                def _(ci):
                    pa_chunk(r, hli, hc, msp, ebase + ci * chunk, chunk,
                             dst_v)
                if etail:
                    pa_chunk(r, hli, hc, msp, ebase + nchunk * chunk,
                             etail, dst_t)

            plsc.subcore_barrier()

            @pl.loop(0, nchunk)
            def _(ci):
                pb_chunk(r, ebase + ci * chunk, chunk, dst_v)
            if etail:
                pb_chunk(r, ebase + nchunk * chunk, etail, dst_t)

            plsc.subcore_barrier()

            for off, n in _stripe_chunks():
                wb_chunk(r, s * stripe + off, n)
            if tail:
                @pl.when(s == NS - 1)
                def _():
                    wb_chunk(r, NS * stripe, tail)
            plsc.subcore_barrier()

    kern = pl.kernel(
        body,
        out_type=[
            jax.ShapeDtypeStruct((2, R, n_tgt, cols), F32),
            jax.ShapeDtypeStruct((th * n_edges,), F32),
        ],
        mesh=_MESH,
        compiler_params=pltpu.CompilerParams(needs_layout_passes=False),
        scratch_types=[
            pltpu.VMEM((n_tgt,), F32),
            pltpu.VMEM((n_tgt,), F32),
            pltpu.VMEM((2, 16), F32),
            pltpu.VMEM((128,), I32),
            pltpu.VMEM((128,), I32),
            pltpu.VMEM((32,), I32),
            pltpu.VMEM((128,), F32),
            pltpu.VMEM((128,), F32),
            pltpu.VMEM((128,), F32),
            pltpu.VMEM((128,), F32),
            pltpu.VMEM((128,), F32),
            pltpu.VMEM((128, cols), F32),
            pltpu.SemaphoreType.DMA,
            pltpu.SemaphoreType.DMA,
            pltpu.VMEM_SHARED((n_tgt, cols), F32),
        ] + [pltpu.VMEM_SHARED((n_tgt,), F32) for _ in range(hl)],
    )
    return kern


_edge_l1 = _make_edge_kernel(N_T0, E0, 2, 64, True)
_edge_l2 = _make_edge_kernel(N_T1, E1, 1, 32, False, row_cols=128)


def _d1_body(o_ref, stats_ref):
    b = pl.program_id(1)
    a = o_ref[0, 0]
    bb = o_ref[1, 0]
    sa = a.sum(0)
    sb = bb.sum(0)
    qa = (a * a).sum(0)
    qb = (bb * bb).sum(0)
    st = jnp.concatenate(
        [sa[None], sb[None], qa[None], qb[None],
         jnp.zeros((4, 128), F32)], axis=0)

    @pl.when(b == 0)
    def _():
        stats_ref[0] = st

    @pl.when(b > 0)
    def _():
        stats_ref[0] = stats_ref[0] + st


def _d1(out1):
    return pl.pallas_call(
        _d1_body,
        grid=(R, NB_B),
        in_specs=[pl.BlockSpec((2, 1, BROWS, 128), lambda r, b: (0, r, b, 0))],
        out_specs=pl.BlockSpec((1, 8, 128), lambda r, b: (r, 0, 0)),
        out_shape=jax.ShapeDtypeStruct((R, 8, 128), F32),
    )(out1)


DROWS = 256
NB_D = N_T1 // DROWS


def _d2_body(o_ref, stats_ref, g_ref, bta_ref, w2_ref, as2_ref, ad2_ref,
             h2_ref, als2_ref, ald2_ref, ms2_ref, md2_ref):
    b = pl.program_id(1)
    x = jnp.concatenate([o_ref[0, 0], o_ref[1, 0]], axis=1)
    st = stats_ref[0]
    mu = jnp.concatenate([st[0], st[1]]) * F32(1.0 / N_T0)
    sq = jnp.concatenate([st[2], st[3]]) * F32(1.0 / N_T0)
    var = sq - mu * mu
    inv = lax.rsqrt(var + F32(1e-5))
    xn = g_ref[0, 0][None] * (x - mu[None]) * inv[None] + bta_ref[0, 0][None]
    h = jnp.where(xn > 0, xn, jnp.exp(xn) - F32(1.0))
    h2 = jnp.dot(h, w2_ref[0], preferred_element_type=F32)
    zpad = jnp.zeros((DROWS, 96), F32)
    h2_ref[0, 0] = jnp.concatenate([h2[:, :32], zpad], axis=1)
    h2_ref[1, 0] = jnp.concatenate([h2[:, 32:], zpad], axis=1)
    als = (h2 * as2_ref[0, 0][None]).sum(-1)
    ald = (h2 * ad2_ref[0, 0][None]).sum(-1)
    z7 = jnp.zeros((DROWS, 7), F32)
    als2_ref[0] = jnp.concatenate([als[:, None], z7], axis=1)
    ald2_ref[0] = jnp.concatenate([ald[:, None], z7], axis=1)
    pad = jnp.full((15,), -1e30, F32)
    cs = jnp.concatenate([als.max()[None], pad])
    cd = jnp.concatenate([ald.max()[None], pad])

    @pl.when(b == 0)
    def _():
        ms2_ref[0, 0] = cs
        md2_ref[0, 0] = cd

    @pl.when(b > 0)
    def _():
        ms2_ref[0, 0] = jnp.maximum(ms2_ref[0, 0], cs)
        md2_ref[0, 0] = jnp.maximum(md2_ref[0, 0], cd)


def _d2(out1, stats, bn_g, bn_b, W2, att_s2, att_d2):
    return pl.pallas_call(
        _d2_body,
        grid=(R, NB_D),
        in_specs=[
            pl.BlockSpec((2, 1, DROWS, 128), lambda r, b: (0, r, b, 0)),
            pl.BlockSpec((1, 8, 128), lambda r, b: (r, 0, 0)),
            pl.BlockSpec((1, 1, HEADS * HID), lambda r, b: (r, 0, 0)),
            pl.BlockSpec((1, 1, HEADS * HID), lambda r, b: (r, 0, 0)),
            pl.BlockSpec((1, HEADS * HID, OUT), lambda r, b: (r, 0, 0)),
            pl.BlockSpec((1, 1, OUT), lambda r, b: (r, 0, 0)),
            pl.BlockSpec((1, 1, OUT), lambda r, b: (r, 0, 0)),
        ],
        out_specs=[
            pl.BlockSpec((2, 1, DROWS, 128), lambda r, b: (0, r, b, 0)),
            pl.BlockSpec((1, DROWS, 8), lambda r, b: (r, b, 0)),
            pl.BlockSpec((1, DROWS, 8), lambda r, b: (r, b, 0)),
            pl.BlockSpec((1, 1, 16), lambda r, b: (r, 0, 0)),
            pl.BlockSpec((1, 1, 16), lambda r, b: (r, 0, 0)),
        ],
        out_shape=[
            jax.ShapeDtypeStruct((2, R, N_T1, 128), F32),
            jax.ShapeDtypeStruct((R, N_T1, 8), F32),
            jax.ShapeDtypeStruct((R, N_T1, 8), F32),
            jax.ShapeDtypeStruct((R, 1, 16), F32),
            jax.ShapeDtypeStruct((R, 1, 16), F32),
        ],
    )(out1, stats, bn_g, bn_b, W2, att_s2, att_d2)


FROWS = 256
NB_F = N_T1 // FROWS


def _f_body(o2_ref, b2f_ref, rlf_ref, wm1_ref, bm1_ref, gm_ref, btm_ref,
            wm2_ref, bm2_ref, out_ref, g_scr, st_scr):
    p = pl.program_id(0)
    b = pl.program_id(1)

    @pl.when(p == 0)
    def _():
        f = jnp.concatenate(
            [o2_ref[0, 0][:, :32], o2_ref[1, 0][:, :32],
             o2_ref[0, 1][:, :32], o2_ref[1, 1][:, :32],
             o2_ref[0, 2][:, :32], o2_ref[1, 2][:, :32]], axis=1)
        f = (f + b2f_ref[0][None]) * rlf_ref[0][None]
        g = jnp.dot(f, wm1_ref[...], preferred_element_type=F32) \
            + bm1_ref[0][None]
        g_scr[pl.ds(b * FROWS, FROWS)] = g
        st = jnp.concatenate(
            [g.sum(0)[None], (g * g).sum(0)[None],
             jnp.zeros((6, 192), F32)], axis=0)

        @pl.when(b == 0)
        def _():
            st_scr[...] = st

        @pl.when(b > 0)
        def _():
            st_scr[...] = st_scr[...] + st

    @pl.when(p == 1)
    def _():
        g = g_scr[pl.ds(b * FROWS, FROWS)]
        mu = st_scr[0] * F32(1.0 / N_T1)
        var = st_scr[1] * F32(1.0 / N_T1) - mu * mu
        inv = lax.rsqrt(var + F32(1e-5))
        gn = gm_ref[0][None] * (g - mu[None]) * inv[None] + btm_ref[0][None]
        gn = jnp.maximum(gn, F32(0.0))
        out_ref[...] = jnp.dot(gn, wm2_ref[...],
                               preferred_element_type=F32) + bm2_ref[0][None]


def _f_stage(out2, b2f, rlf, Wm1, bm1, gm, betam, Wm2, bm2):
    return pl.pallas_call(
        _f_body,
        grid=(2, NB_F),
        in_specs=[
            pl.BlockSpec((2, R, FROWS, 128), lambda p, b: (0, 0, b, 0)),
            pl.BlockSpec((1, R * OUT), lambda p, b: (0, 0)),
            pl.BlockSpec((1, R * OUT), lambda p, b: (0, 0)),
            pl.BlockSpec((R * OUT, R * OUT), lambda p, b: (0, 0)),
            pl.BlockSpec((1, R * OUT), lambda p, b: (0, 0)),
            pl.BlockSpec((1, R * OUT), lambda p, b: (0, 0)),
            pl.BlockSpec((1, R * OUT), lambda p, b: (0, 0)),
            pl.BlockSpec((R * OUT, OUT), lambda p, b: (0, 0)),
            pl.BlockSpec((1, OUT), lambda p, b: (0, 0)),
        ],
        out_specs=pl.BlockSpec((FROWS, OUT), lambda p, b: (b, 0)),
        out_shape=jax.ShapeDtypeStruct((N_T1, OUT), F32),
        scratch_shapes=[
            pltpu.VMEM((N_T1, R * OUT), F32),
            pltpu.VMEM((8, R * OUT), F32),
        ],
    )(out2, b2f, rlf, Wm1, bm1, gm, betam, Wm2, bm2)


def kernel(x, n_ids, ei0, ei1, RL_thresholds, W1, att_s1, att_d1, b1, bn_g,
           bn_b, W2, att_s2, att_d2, b2, Wm1, bm1, gm, betam, Wm2, bm2):
    idx = n_ids[:, :N_T0].reshape(-1).astype(I32)
    idx = jnp.concatenate([idx, jnp.zeros((NGPAD - R * N_T0,), I32)])
    src0 = ei0[:, 0, :].reshape(-1).astype(I32)
    dst0 = ei0[:, 1, :].reshape(-1).astype(I32)
    src1 = ei1[:, 0, :].reshape(-1).astype(I32)
    dst1 = ei1[:, 1, :].reshape(-1).astype(I32)

    xs = _gather_rows(x, idx)
    xs3 = xs[: R * N_T0].reshape(R, N_T0, D_IN)

    H1, als1, ald1, ms1, md1 = _mm1(xs3, W1, att_s1, att_d1)

    alsf1 = als1.transpose(2, 0, 1).reshape(-1)
    aldf1 = ald1.transpose(2, 0, 1).reshape(-1)
    out1, _ = _edge_l1(H1, alsf1, aldf1, ms1, md1, src0, dst0)

    stats = _d1(out1)
    H2, als2, ald2, ms2, md2 = _d2(out1, stats, bn_g[:, None], bn_b[:, None],
                                   W2, att_s2, att_d2)

    alsf2 = als2[:, :, 0].reshape(-1)
    aldf2 = ald2[:, :, 0].reshape(-1)
    out2, _ = _edge_l2(H2, alsf2, aldf2, ms2, md2, src1, dst1)

    b2f = b2.reshape(1, R * OUT)
    rlf = jnp.repeat(RL_thresholds[:, 0], OUT).reshape(1, R * OUT)
    return _f_stage(out2, b2f, rlf, Wm1, bm1.reshape(1, -1), gm.reshape(1, -1),
                    betam.reshape(1, -1), Wm2, bm2.reshape(1, -1))
